# Initial kernel scaffold; baseline (speedup 1.0000x reference)
import jax
import jax.numpy as jnp
from jax.experimental import pallas as pl


def kernel(x, table, W, b):
    raise NotImplementedError("write your pallas kernel here")


# speedup vs baseline: 1.2828x; 1.2828x over previous
import functools

import jax
import jax.numpy as jnp
from jax import lax
from jax.experimental import pallas as pl
from jax.experimental.pallas import tpu as pltpu
from jax.experimental.pallas import tpu_sc as plsc

EMBED = 64
CHUNK = 640


def _sc_gather(table, idx):
    n = idx.shape[0]
    info = plsc.get_sparse_core_info()
    nc, ns = info.num_cores, info.num_subcores
    nw = nc * ns
    per_w = n // nw
    n_chunks = per_w // CHUNK
    assert per_w % CHUNK == 0 and n % nw == 0

    mesh = plsc.VectorSubcoreMesh(core_axis_name="c", subcore_axis_name="s")

    @functools.partial(
        pl.kernel,
        mesh=mesh,
        out_type=jax.ShapeDtypeStruct((n, EMBED), jnp.float32),
        scratch_types=[
            pltpu.VMEM((CHUNK,), jnp.int32),
            pltpu.VMEM((CHUNK, EMBED), jnp.float32),
            pltpu.SemaphoreType.DMA,
        ],
        compiler_params=pltpu.CompilerParams(use_tc_tiling_on_sc=False),
    )
    def gather_kernel(table_hbm, idx_hbm, out_hbm, idx_v, rows_v, sem):
        wid = lax.axis_index("s") * nc + lax.axis_index("c")
        base = wid * per_w

        def body(i, carry):
            off = base + i * CHUNK
            pltpu.sync_copy(idx_hbm.at[pl.ds(off, CHUNK)], idx_v)
            pltpu.async_copy(table_hbm.at[idx_v], rows_v, sem).wait()
            pltpu.sync_copy(rows_v, out_hbm.at[pl.ds(off, CHUNK)])
            return carry

        lax.fori_loop(0, n_chunks, body, 0)

    return gather_kernel(table, idx)


def _tc_project(a, w, bias):
    m, kdim = a.shape
    ndim = w.shape[1]
    bm = 2048
    assert m % bm == 0

    def body(a_ref, w_ref, b_ref, o_ref):
        acc = jnp.dot(a_ref[...], w_ref[...], preferred_element_type=jnp.float32)
        o_ref[...] = jnp.maximum(acc + b_ref[...], 0.0)

    return pl.pallas_call(
        body,
        grid=(m // bm,),
        in_specs=[
            pl.BlockSpec((bm, kdim), lambda i: (i, 0)),
            pl.BlockSpec((kdim, ndim), lambda i: (0, 0)),
            pl.BlockSpec((1, ndim), lambda i: (0, 0)),
        ],
        out_specs=pl.BlockSpec((bm, ndim), lambda i: (i, 0)),
        out_shape=jax.ShapeDtypeStruct((m, ndim), jnp.float32),
        compiler_params=pltpu.CompilerParams(
            dimension_semantics=("arbitrary",),
        ),
    )(a, w, bias.reshape(1, ndim))


def kernel(x, table, W, b):
    bs, seq, k = x.shape
    idx = x.reshape(-1)
    gathered = _sc_gather(table, idx)
    a = gathered.reshape(bs * seq, k * EMBED)
    out = _tc_project(a, W, b)
    return out.reshape(bs, seq, EMBED)


# speedup vs baseline: 1.3478x; 1.0507x over previous
import functools

import jax
import jax.numpy as jnp
from jax import lax
from jax.experimental import pallas as pl
from jax.experimental.pallas import tpu as pltpu
from jax.experimental.pallas import tpu_sc as plsc

EMBED = 64


def _sc_gather(table, x):
    bs, seqk = x.shape
    info = plsc.get_sparse_core_info()
    nc, ns = info.num_cores, info.num_subcores
    nw = nc * ns
    b_per_w = bs // nw
    assert bs % nw == 0

    mesh = plsc.VectorSubcoreMesh(core_axis_name="c", subcore_axis_name="s")

    @functools.partial(
        pl.kernel,
        mesh=mesh,
        out_type=jax.ShapeDtypeStruct((bs, seqk, EMBED), jnp.float32),
        scratch_types=[
            pltpu.VMEM((seqk,), jnp.int32),
            pltpu.VMEM((seqk, EMBED), jnp.float32),
            pltpu.SemaphoreType.DMA,
        ],
        compiler_params=pltpu.CompilerParams(use_tc_tiling_on_sc=False),
    )
    def gather_kernel(table_hbm, x_hbm, out_hbm, idx_v, rows_v, sem):
        wid = lax.axis_index("s") * nc + lax.axis_index("c")
        base = wid * b_per_w

        def body(i, carry):
            b = base + i
            pltpu.sync_copy(x_hbm.at[b], idx_v)
            pltpu.async_copy(table_hbm.at[idx_v], rows_v, sem).wait()
            pltpu.sync_copy(rows_v, out_hbm.at[b])
            return carry

        lax.fori_loop(0, b_per_w, body, 0)

    return gather_kernel(table, x)


def _tc_project(a, w, bias):
    m, kdim = a.shape
    ndim = w.shape[1]
    bm = 2048
    assert m % bm == 0

    def body(a_ref, w_ref, b_ref, o_ref):
        acc = jnp.dot(a_ref[...], w_ref[...], preferred_element_type=jnp.float32)
        o_ref[...] = jnp.maximum(acc + b_ref[...], 0.0)

    return pl.pallas_call(
        body,
        grid=(m // bm,),
        in_specs=[
            pl.BlockSpec((bm, kdim), lambda i: (i, 0)),
            pl.BlockSpec((kdim, ndim), lambda i: (0, 0)),
            pl.BlockSpec((1, ndim), lambda i: (0, 0)),
        ],
        out_specs=pl.BlockSpec((bm, ndim), lambda i: (i, 0)),
        out_shape=jax.ShapeDtypeStruct((m, ndim), jnp.float32),
        compiler_params=pltpu.CompilerParams(
            dimension_semantics=("arbitrary",),
        ),
    )(a, w, bias.reshape(1, ndim))


def kernel(x, table, W, b):
    bs, seq, k = x.shape
    gathered = _sc_gather(table, x.reshape(bs, seq * k))
    a = gathered.reshape(bs * seq, k * EMBED)
    out = _tc_project(a, W, b)
    return out.reshape(bs, seq, EMBED)


# speedup vs baseline: 1.5157x; 1.1246x over previous
import functools

import jax
import jax.numpy as jnp
from jax import lax
from jax.experimental import pallas as pl
from jax.experimental.pallas import tpu as pltpu
from jax.experimental.pallas import tpu_sc as plsc

EMBED = 64


def _sc_gather(table, x):
    bs, seqk = x.shape
    info = plsc.get_sparse_core_info()
    nc, ns = info.num_cores, info.num_subcores
    nw = nc * ns
    b_per_w = bs // nw
    assert bs % nw == 0 and seqk % 2 == 0
    half = seqk // 2

    mesh = plsc.VectorSubcoreMesh(core_axis_name="c", subcore_axis_name="s")

    @functools.partial(
        pl.kernel,
        mesh=mesh,
        out_type=jax.ShapeDtypeStruct((bs * seqk, EMBED), jnp.float32),
        scratch_types=[
            pltpu.VMEM((seqk,), jnp.int32),
            pltpu.VMEM((seqk, EMBED), jnp.float32),
            pltpu.SemaphoreType.DMA,
        ],
        compiler_params=pltpu.CompilerParams(use_tc_tiling_on_sc=False),
    )
    def gather_kernel(table_hbm, x_hbm, out_hbm, idx_v, rows_v, sem):
        wid = lax.axis_index("s") * nc + lax.axis_index("c")
        base = wid * b_per_w

        def body(i, carry):
            b = base + i
            pltpu.sync_copy(x_hbm.at[b], idx_v)
            pltpu.async_copy(table_hbm.at[idx_v], rows_v, sem).wait()
            pltpu.sync_copy(rows_v, out_hbm.at[pl.ds(b * seqk, seqk)])
            return carry

        lax.fori_loop(0, b_per_w, body, 0)

    return gather_kernel(table, x)


def _tc_project(g2, wfull, bias2, n_tok):
    m5 = g2.shape[0]
    bm5 = 2560
    assert m5 % bm5 == 0
    grid = m5 // bm5
    bp = bm5 // 5

    def body(g_ref, w_ref, b_ref, o_ref):
        xg = g_ref[...].reshape(bp, 5 * 128)
        acc = jnp.dot(xg, w_ref[...], preferred_element_type=jnp.float32)
        o_ref[...] = jnp.maximum(acc + b_ref[...], 0.0)

    return pl.pallas_call(
        body,
        grid=(grid,),
        in_specs=[
            pl.BlockSpec((bm5, 128), lambda i: (i, 0)),
            pl.BlockSpec((5 * 128, 128), lambda i: (0, 0)),
            pl.BlockSpec((1, 128), lambda i: (0, 0)),
        ],
        out_specs=pl.BlockSpec((bp, 128), lambda i: (i, 0)),
        out_shape=jax.ShapeDtypeStruct((n_tok // 2, 128), jnp.float32),
        compiler_params=pltpu.CompilerParams(
            dimension_semantics=("arbitrary",),
        ),
    )(g2, wfull, bias2)


def kernel(x, table, W, b):
    bs, seq, k = x.shape
    ke = k * EMBED
    g = _sc_gather(table, x.reshape(bs, seq * k))
    g2 = g.reshape(bs * seq * k // 2, 2 * EMBED)
    z = jnp.zeros((ke, EMBED), jnp.float32)
    wfull = jnp.concatenate(
        [jnp.concatenate([W, z], axis=1), jnp.concatenate([z, W], axis=1)],
        axis=0,
    )
    bias2 = jnp.concatenate([b, b]).reshape(1, 2 * EMBED)
    out2 = _tc_project(g2, wfull, bias2, bs * seq)
    return out2.reshape(bs, seq, EMBED)


# speedup vs baseline: 1.6011x; 1.0564x over previous
import functools

import jax
import jax.numpy as jnp
from jax import lax
from jax.experimental import pallas as pl
from jax.experimental.pallas import tpu as pltpu
from jax.experimental.pallas import tpu_sc as plsc

EMBED = 64


def _sc_gather(table, x):
    bs, seqk = x.shape
    info = plsc.get_sparse_core_info()
    nc, ns = info.num_cores, info.num_subcores
    nw = nc * ns
    b_per_w = bs // nw
    assert bs % nw == 0 and seqk % 2 == 0
    half = seqk // 2

    mesh = plsc.VectorSubcoreMesh(core_axis_name="c", subcore_axis_name="s")

    @functools.partial(
        pl.kernel,
        mesh=mesh,
        out_type=jax.ShapeDtypeStruct((bs * seqk, EMBED), jnp.float32),
        scratch_types=[
            pltpu.VMEM((seqk,), jnp.int32),
            pltpu.VMEM((seqk, EMBED), jnp.float32),
            pltpu.SemaphoreType.DMA,
        ],
        compiler_params=pltpu.CompilerParams(use_tc_tiling_on_sc=False),
    )
    def gather_kernel(table_hbm, x_hbm, out_hbm, idx_v, rows_v, sem):
        wid = lax.axis_index("s") * nc + lax.axis_index("c")
        base = wid * b_per_w

        def body(i, carry):
            b = base + i
            pltpu.sync_copy(x_hbm.at[b], idx_v)
            pltpu.async_copy(table_hbm.at[idx_v], rows_v, sem).wait()
            pltpu.sync_copy(rows_v, out_hbm.at[pl.ds(b * seqk, seqk)])
            return carry

        lax.fori_loop(0, b_per_w, body, 0)

    return gather_kernel(table, x)


def _tc_project(g2, wfull, bias2, n_tok):
    m5 = g2.shape[0]
    bm5 = 2560
    assert m5 % bm5 == 0
    grid = m5 // bm5
    bp = bm5 // 5

    def body(g_ref, w_ref, b_ref, o_ref):
        xg = g_ref[...].reshape(bp, 5 * 128)
        acc = jnp.dot(xg, w_ref[...], preferred_element_type=jnp.float32)
        o_ref[...] = jnp.maximum(acc + b_ref[...], 0.0)

    return pl.pallas_call(
        body,
        grid=(grid,),
        in_specs=[
            pl.BlockSpec((bm5, 128), lambda i: (i, 0)),
            pl.BlockSpec((5 * 128, 128), lambda i: (0, 0)),
            pl.BlockSpec((1, 128), lambda i: (0, 0)),
        ],
        out_specs=pl.BlockSpec((bp, 128), lambda i: (i, 0)),
        out_shape=jax.ShapeDtypeStruct((n_tok // 2, 128), jnp.float32),
        compiler_params=pltpu.CompilerParams(
            dimension_semantics=("arbitrary",),
        ),
    )(g2, wfull, bias2)


def kernel(x, table, W, b):
    bs, seq, k = x.shape
    ke = k * EMBED
    v = table.shape[0]
    tablep = jnp.pad(table, ((0, 0), (0, EMBED))).reshape(2 * v, EMBED)
    g = _sc_gather(tablep, (x * 2).reshape(bs, seq * k))
    g2 = g.reshape(bs * seq * k // 2, 2 * EMBED)
    z = jnp.zeros((ke, EMBED), jnp.float32)
    wfull = jnp.concatenate(
        [jnp.concatenate([W, z], axis=1), jnp.concatenate([z, W], axis=1)],
        axis=0,
    )
    bias2 = jnp.concatenate([b, b]).reshape(1, 2 * EMBED)
    out2 = _tc_project(g2, wfull, bias2, bs * seq)
    return out2.reshape(bs, seq, EMBED)
